# baseline (speedup 1.0000x reference)
import functools

import jax
import jax.numpy as jnp
from jax.experimental import pallas as pl
from jax.experimental.pallas import tpu as pltpu

N = 10000
E = 320000
D = 128
ED = 16


def _p_kernel(x_ref, w1t_ref, o_ref):
    o_ref[...] = jnp.dot(x_ref[...], w1t_ref[...],
                         preferred_element_type=jnp.float32)


def _q_kernel(ea_ref, w2t_ref, b_ref, o_ref):
    o_ref[...] = jnp.dot(ea_ref[...], w2t_ref[...],
                         preferred_element_type=jnp.float32) + b_ref[...]


def _out_kernel(x_ref, aggr_ref, wnt_ref, wwt_ref, b_ref, o_ref):
    acc = jnp.dot(x_ref[...], wnt_ref[...], preferred_element_type=jnp.float32)
    acc += jnp.dot(aggr_ref[...], wwt_ref[...], preferred_element_type=jnp.float32)
    o_ref[...] = jnp.tanh(acc + b_ref[...])


def kernel(x, edge_index, edge_attr, W_node_w, W_node_b, W_neigh_w, W_neigh_b,
           W_att_w, W_att_b):
    row = edge_index[0]
    col = edge_index[1]
    W1t = W_att_w[:, :D].T
    W2t = W_att_w[:, D:].T

    P = pl.pallas_call(
        _p_kernel,
        out_shape=jax.ShapeDtypeStruct((N, D), jnp.float32),
    )(x, W1t)

    QB = 8000
    Q = pl.pallas_call(
        _q_kernel,
        grid=(E // QB,),
        in_specs=[
            pl.BlockSpec((QB, ED), lambda i: (i, 0)),
            pl.BlockSpec((ED, D), lambda i: (0, 0)),
            pl.BlockSpec((1, D), lambda i: (0, 0)),
        ],
        out_specs=pl.BlockSpec((QB, D), lambda i: (i, 0)),
        out_shape=jax.ShapeDtypeStruct((E, D), jnp.float32),
    )(edge_attr, W2t, W_att_b.reshape(1, D))

    x_col = jnp.take(x, col, axis=0)
    alpha = jax.nn.sigmoid(jnp.take(P, col, axis=0) + Q)
    aggr = jax.ops.segment_sum(x_col * alpha, row, num_segments=N)

    out = pl.pallas_call(
        _out_kernel,
        out_shape=jax.ShapeDtypeStruct((N, D), jnp.float32),
    )(x, aggr, W_node_w.T, W_neigh_w.T,
      (W_node_b + W_neigh_b).reshape(1, D))
    return out


# speedup vs baseline: 1.2212x; 1.2212x over previous
import functools

import jax
import jax.numpy as jnp
from jax import lax
from jax.experimental import pallas as pl
from jax.experimental.pallas import tpu as pltpu
from jax.experimental.pallas import tpu_sc as plsc

N = 10000
E = 320000
D = 128
ED = 16

NC = 2
NS = 16
NW = NC * NS
EPW = E // NW
C = 80
NCHUNK = EPW // C
ZCH = 40
NZ = N // ZCH


def _g_kernel(x_ref, w1t_ref, g_ref):
    g_ref[:, :D] = x_ref[...]
    g_ref[:, D:] = jnp.dot(x_ref[...], w1t_ref[...],
                           preferred_element_type=jnp.float32)


def _q_kernel(ea_ref, w2t_ref, b_ref, o_ref):
    o_ref[...] = jnp.dot(ea_ref[...], w2t_ref[...],
                         preferred_element_type=jnp.float32) + b_ref[...]


def _out_kernel(x_ref, a0_ref, a1_ref, wnt_ref, wwt_ref, b_ref, o_ref):
    acc = jnp.dot(x_ref[...], wnt_ref[...], preferred_element_type=jnp.float32)
    aggr = a0_ref[...] + a1_ref[...]
    acc += jnp.dot(aggr, wwt_ref[...], preferred_element_type=jnp.float32)
    o_ref[...] = jnp.tanh(acc + b_ref[...])


def _sc_body(g_hbm, q_hbm, col_hbm, row_hbm, out_hbm,
             colv, rowv, gv, qv, zbuf, aggr_sh, gsem, qsem):
    c = lax.axis_index("c")
    s = lax.axis_index("s")
    wid = c * NS + s
    ebase = wid * EPW

    def _zero_row(r, _):
        for g in range(D // 16):
            zbuf[r, pl.ds(16 * g, 16)] = jnp.zeros((16,), jnp.float32)
        return _
    lax.fori_loop(0, ZCH, _zero_row, None)
    for k in range(pl.cdiv(NZ, NS)):
        t = s + k * NS
        if (k + 1) * NS <= NZ:
            pltpu.sync_copy(zbuf, aggr_sh.at[pl.ds(t * ZCH, ZCH)])
        else:
            @pl.when(t < NZ)
            def _():
                pltpu.sync_copy(zbuf, aggr_sh.at[pl.ds(t * ZCH, ZCH)])
    plsc.subcore_barrier()

    def _chunk(j, _):
        base = ebase + j * C
        pltpu.sync_copy(col_hbm.at[pl.ds(base, C)], colv)
        pltpu.sync_copy(row_hbm.at[pl.ds(base, C)], rowv)
        gcp = pltpu.async_copy(g_hbm.at[colv], gv, gsem)
        qcp = pltpu.async_copy(q_hbm.at[pl.ds(base, C)], qv, qsem)
        gcp.wait()
        qcp.wait()

        def _edge(r, _):
            for g in range(D // 16):
                xval = gv[r, pl.ds(16 * g, 16)]
                pval = gv[r, pl.ds(D + 16 * g, 16)]
                qval = qv[r, pl.ds(16 * g, 16)]
                a = 1.0 / (1.0 + jnp.exp(-(pval + qval)))
                qv[r, pl.ds(16 * g, 16)] = xval * a
            return _
        lax.fori_loop(0, C, _edge, None)
        pltpu.sync_copy(qv, aggr_sh.at[rowv], add=True)
        return _
    lax.fori_loop(0, NCHUNK, _chunk, None)
    plsc.subcore_barrier()

    for k in range(pl.cdiv(NZ, NS)):
        t = s + k * NS

        def _copy_out(t=t):
            pltpu.sync_copy(aggr_sh.at[pl.ds(t * ZCH, ZCH)], zbuf)
            pltpu.sync_copy(zbuf, out_hbm.at[c, pl.ds(t * ZCH, ZCH)])
        if (k + 1) * NS <= NZ:
            _copy_out()
        else:
            pl.when(t < NZ)(_copy_out)


_sc_scatter = functools.partial(
    pl.kernel,
    out_type=jax.ShapeDtypeStruct((NC, N, D), jnp.float32),
    mesh=plsc.VectorSubcoreMesh(core_axis_name="c", subcore_axis_name="s"),
    scratch_types=[
        pltpu.VMEM((C,), jnp.int32),
        pltpu.VMEM((C,), jnp.int32),
        pltpu.VMEM((C, 2 * D), jnp.float32),
        pltpu.VMEM((C, D), jnp.float32),
        pltpu.VMEM((ZCH, D), jnp.float32),
        pltpu.VMEM_SHARED((N, D), jnp.float32),
        pltpu.SemaphoreType.DMA,
        pltpu.SemaphoreType.DMA,
    ],
)(_sc_body)


def kernel(x, edge_index, edge_attr, W_node_w, W_node_b, W_neigh_w, W_neigh_b,
           W_att_w, W_att_b):
    row = edge_index[0]
    col = edge_index[1]
    W1t = W_att_w[:, :D].T
    W2t = W_att_w[:, D:].T

    G = pl.pallas_call(
        _g_kernel,
        out_shape=jax.ShapeDtypeStruct((N, 2 * D), jnp.float32),
    )(x, W1t)

    QB = 8000
    Q = pl.pallas_call(
        _q_kernel,
        grid=(E // QB,),
        in_specs=[
            pl.BlockSpec((QB, ED), lambda i: (i, 0)),
            pl.BlockSpec((ED, D), lambda i: (0, 0)),
            pl.BlockSpec((1, D), lambda i: (0, 0)),
        ],
        out_specs=pl.BlockSpec((QB, D), lambda i: (i, 0)),
        out_shape=jax.ShapeDtypeStruct((E, D), jnp.float32),
    )(edge_attr, W2t, W_att_b.reshape(1, D))

    aggr_parts = _sc_scatter(G, Q, col, row)

    out = pl.pallas_call(
        _out_kernel,
        out_shape=jax.ShapeDtypeStruct((N, D), jnp.float32),
    )(x, aggr_parts[0], aggr_parts[1], W_node_w.T, W_neigh_w.T,
      (W_node_b + W_neigh_b).reshape(1, D))
    return out


# speedup vs baseline: 3.3752x; 2.7638x over previous
import functools

import jax
import jax.numpy as jnp
from jax import lax
from jax.experimental import pallas as pl
from jax.experimental.pallas import tpu as pltpu
from jax.experimental.pallas import tpu_sc as plsc

N = 10000
E = 320000
D = 128
ED = 16

NC = 2
NS = 16
NW = NC * NS
EPW = E // NW
C = 80
NCHUNK = EPW // C
ZCH = 40
NZ = N // ZCH


def _g_kernel(x_ref, w1t_ref, g_ref):
    g_ref[:, :D] = x_ref[...]
    g_ref[:, D:] = -jnp.dot(x_ref[...], w1t_ref[...],
                            preferred_element_type=jnp.float32)


def _q_kernel(ea_ref, w2t_ref, b_ref, o_ref):
    o_ref[...] = -(jnp.dot(ea_ref[...], w2t_ref[...],
                           preferred_element_type=jnp.float32) + b_ref[...])


def _out_kernel(x_ref, a0_ref, a1_ref, wnt_ref, wwt_ref, b_ref, o_ref):
    acc = jnp.dot(x_ref[...], wnt_ref[...], preferred_element_type=jnp.float32)
    aggr = a0_ref[...] + a1_ref[...]
    acc += jnp.dot(aggr, wwt_ref[...], preferred_element_type=jnp.float32)
    o_ref[...] = jnp.tanh(acc + b_ref[...])


def _sc_body(g_hbm, q_hbm, col_hbm, row_hbm, out_hbm,
             colv, rowv, gv, qv, zbuf, aggr_sh, gsem, qsem):
    c = lax.axis_index("c")
    s = lax.axis_index("s")
    wid = c * NS + s
    ebase = wid * EPW

    @plsc.parallel_loop(0, ZCH, unroll=4)
    def _zero_row(r):
        for g in range(D // 16):
            zbuf[r, pl.ds(16 * g, 16)] = jnp.zeros((16,), jnp.float32)
    for k in range(pl.cdiv(NZ, NS)):
        t = s + k * NS
        if (k + 1) * NS <= NZ:
            pltpu.sync_copy(zbuf, aggr_sh.at[pl.ds(t * ZCH, ZCH)])
        else:
            @pl.when(t < NZ)
            def _():
                pltpu.sync_copy(zbuf, aggr_sh.at[pl.ds(t * ZCH, ZCH)])
    plsc.subcore_barrier()

    def _chunk(j, _):
        base = ebase + j * C
        pltpu.sync_copy(col_hbm.at[pl.ds(base, C)], colv)
        pltpu.sync_copy(row_hbm.at[pl.ds(base, C)], rowv)
        gcp = pltpu.async_copy(g_hbm.at[colv], gv, gsem)
        qcp = pltpu.async_copy(q_hbm.at[pl.ds(base, C)], qv, qsem)
        gcp.wait()
        qcp.wait()

        @plsc.parallel_loop(0, C, unroll=4)
        def _edge(r):
            for g in range(D // 16):
                xval = gv[r, pl.ds(16 * g, 16)]
                pval = gv[r, pl.ds(D + 16 * g, 16)]
                qval = qv[r, pl.ds(16 * g, 16)]
                a = 1.0 / (1.0 + jnp.exp(pval + qval))
                qv[r, pl.ds(16 * g, 16)] = xval * a
        pltpu.sync_copy(qv, aggr_sh.at[rowv], add=True)
        return _
    lax.fori_loop(0, NCHUNK, _chunk, None)
    plsc.subcore_barrier()

    for k in range(pl.cdiv(NZ, NS)):
        t = s + k * NS

        def _copy_out(t=t):
            pltpu.sync_copy(aggr_sh.at[pl.ds(t * ZCH, ZCH)], zbuf)
            pltpu.sync_copy(zbuf, out_hbm.at[c, pl.ds(t * ZCH, ZCH)])
        if (k + 1) * NS <= NZ:
            _copy_out()
        else:
            pl.when(t < NZ)(_copy_out)


_sc_scatter = functools.partial(
    pl.kernel,
    out_type=jax.ShapeDtypeStruct((NC, N, D), jnp.float32),
    mesh=plsc.VectorSubcoreMesh(core_axis_name="c", subcore_axis_name="s"),
    scratch_types=[
        pltpu.VMEM((C,), jnp.int32),
        pltpu.VMEM((C,), jnp.int32),
        pltpu.VMEM((C, 2 * D), jnp.float32),
        pltpu.VMEM((C, D), jnp.float32),
        pltpu.VMEM((ZCH, D), jnp.float32),
        pltpu.VMEM_SHARED((N, D), jnp.float32),
        pltpu.SemaphoreType.DMA,
        pltpu.SemaphoreType.DMA,
    ],
)(_sc_body)


def kernel(x, edge_index, edge_attr, W_node_w, W_node_b, W_neigh_w, W_neigh_b,
           W_att_w, W_att_b):
    row = edge_index[0]
    col = edge_index[1]
    W1t = W_att_w[:, :D].T
    W2t = W_att_w[:, D:].T

    G = pl.pallas_call(
        _g_kernel,
        out_shape=jax.ShapeDtypeStruct((N, 2 * D), jnp.float32),
    )(x, W1t)

    QB = 8000
    Q = pl.pallas_call(
        _q_kernel,
        grid=(E // QB,),
        in_specs=[
            pl.BlockSpec((QB, ED), lambda i: (i, 0)),
            pl.BlockSpec((ED, D), lambda i: (0, 0)),
            pl.BlockSpec((1, D), lambda i: (0, 0)),
        ],
        out_specs=pl.BlockSpec((QB, D), lambda i: (i, 0)),
        out_shape=jax.ShapeDtypeStruct((E, D), jnp.float32),
    )(edge_attr, W2t, W_att_b.reshape(1, D))

    aggr_parts = _sc_scatter(G, Q, col, row)

    out = pl.pallas_call(
        _out_kernel,
        out_shape=jax.ShapeDtypeStruct((N, D), jnp.float32),
    )(x, aggr_parts[0], aggr_parts[1], W_node_w.T, W_neigh_w.T,
      (W_node_b + W_neigh_b).reshape(1, D))
    return out
